# baseline (speedup 1.0000x reference)
import functools

import jax
import jax.numpy as jnp
from jax import lax
from jax.experimental import pallas as pl
from jax.experimental.pallas import tpu as pltpu
from jax.experimental.pallas import tpu_sc as plsc

K = 1024
D = 768
B = 4096
TEMP = 1.0
LR = 0.001

BM = 256
BC = 1024

N_SUB = 16
NCOLS = D // N_SUB
RCHUNK = 1024


def _assign_body(dp_ref, c_ref, idx_ref):
    c = c_ref[...]
    dp = dp_ref[...]
    dots = lax.dot_general(dp.astype(jnp.bfloat16), c.astype(jnp.bfloat16),
                           (((1,), (1,)), ((), ())),
                           preferred_element_type=jnp.float32)
    cn2 = lax.dot_general(jnp.ones((8, D), jnp.float32), c * c,
                          (((1,), (1,)), ((), ())),
                          preferred_element_type=jnp.float32,
                          precision=lax.Precision.HIGHEST)
    xn2 = jnp.sum(dp * dp, axis=1, keepdims=True)
    q = xn2 * cn2[0:1, :]
    r = lax.rsqrt(q)
    r = r * (1.5 - 0.5 * q * r * r)
    r = r * (1.5 - 0.5 * q * r * r)
    scale = jnp.where(q < 1e-16, 1e8, r) / TEMP
    score = dots * scale
    m = jnp.max(score, axis=1, keepdims=True)
    iot = lax.broadcasted_iota(jnp.int32, (BM, K), 1)
    cand = jnp.where(score == m, iot, K)
    idx_ref[...] = jnp.min(cand, axis=1, keepdims=True)


def _cluster_body(rows_ref, cols_ref, out_ref):
    bi = pl.program_id(0)
    bj = pl.program_id(1)
    r = rows_ref[...]
    cvec = cols_ref[...]
    eq = jnp.where(r == cvec[None, :], 1.0, 0.0)

    @pl.when(bi == bj)
    def _():
        ii = lax.broadcasted_iota(jnp.int32, (BC, BC), 0)
        jj = lax.broadcasted_iota(jnp.int32, (BC, BC), 1)
        out_ref[...] = jnp.where(ii == jj, 0.0, eq)

    @pl.when(bi != bj)
    def _():
        out_ref[...] = eq


def _sc_scatter_body(dpt_hbm, idx_hbm, zero_hbm, datat_hbm, idx_v, rows_v, acc):
    sid = lax.axis_index("s")
    c0 = sid * NCOLS
    pltpu.sync_copy(zero_hbm.at[:], acc)
    pltpu.sync_copy(idx_hbm.at[:], idx_v)
    ci = lax.broadcasted_iota(jnp.int32, (16,), 0)
    for chunk in range(B // RCHUNK):
        base = chunk * RCHUNK
        pltpu.sync_copy(dpt_hbm.at[pl.ds(c0, NCOLS), pl.ds(base, RCHUNK)],
                        rows_v)

        def _group(g, _):
            kvec = idx_v[pl.ds(base + g * 16, 16)]
            for j in range(16):
                krow = jnp.take(kvec, jnp.full((16,), j, jnp.int32),
                                mode="fill")
                rsplat = jnp.full((16,), g * 16 + j, jnp.int32)
                for c3 in range(NCOLS // 16):
                    vals = plsc.load_gather(rows_v, [ci + (c3 * 16), rsplat])
                    plsc.addupdate_scatter(acc, [ci + (c3 * 16), krow], vals)
            return 0

        lax.fori_loop(0, RCHUNK // 16, _group, 0)
    pltpu.sync_copy(acc, datat_hbm.at[pl.ds(c0, NCOLS)])


def _final_body(datat_ref, c_ref, idx_ref, out_ref):
    iota_k = lax.broadcasted_iota(jnp.int32, (K, 512), 0)
    cnt = jnp.zeros((K, 1), jnp.float32)
    for j in range(B // 512):
        row = idx_ref[j, :]
        eq = jnp.where(iota_k == row[None, :], 1.0, 0.0)
        cnt = cnt + jnp.sum(eq, axis=1, keepdims=True)
    coef = (1.0 - LR) + LR * jnp.where(cnt == 0.0, 1.0, 0.0)
    data = jnp.transpose(datat_ref[...], (1, 0))
    out_ref[...] = LR * data + coef * c_ref[...]


def _assign(datapoints, centroid):
    return pl.pallas_call(
        _assign_body,
        grid=(B // BM,),
        in_specs=[pl.BlockSpec((BM, D), lambda i: (i, 0)),
                  pl.BlockSpec((K, D), lambda i: (0, 0))],
        out_specs=pl.BlockSpec((BM, 1), lambda i: (i, 0)),
        out_shape=jax.ShapeDtypeStruct((B, 1), jnp.int32),
    )(datapoints, centroid)


def _cluster(idx2d, idx1d):
    return pl.pallas_call(
        _cluster_body,
        grid=(B // BC, B // BC),
        in_specs=[pl.BlockSpec((BC, 1), lambda i, j: (i, 0)),
                  pl.BlockSpec((BC,), lambda i, j: (j,))],
        out_specs=pl.BlockSpec((BC, BC), lambda i, j: (i, j)),
        out_shape=jax.ShapeDtypeStruct((B, B), jnp.float32),
    )(idx2d, idx1d)


@functools.partial(
    pl.kernel,
    out_type=jax.ShapeDtypeStruct((D, K), jnp.float32),
    mesh=plsc.VectorSubcoreMesh(core_axis_name="c", subcore_axis_name="s",
                                num_cores=1, num_subcores=N_SUB),
    compiler_params=pltpu.CompilerParams(use_tc_tiling_on_sc=False,
                                         needs_layout_passes=False),
    scratch_types=[
        pltpu.VMEM((B,), jnp.int32),
        pltpu.VMEM((NCOLS, RCHUNK), jnp.float32),
        pltpu.VMEM((NCOLS, K), jnp.float32),
    ],
)
def _sc_scatter(dpt_hbm, idx_hbm, zero_hbm, datat_hbm, idx_v, rows_v, acc):
    _sc_scatter_body(dpt_hbm, idx_hbm, zero_hbm, datat_hbm, idx_v, rows_v, acc)


def _final(datat, centroid, idx8):
    return pl.pallas_call(
        _final_body,
        in_specs=[pl.BlockSpec((D, K), lambda: (0, 0)),
                  pl.BlockSpec((K, D), lambda: (0, 0)),
                  pl.BlockSpec((B // 512, 512), lambda: (0, 0))],
        out_specs=pl.BlockSpec((K, D), lambda: (0, 0)),
        out_shape=jax.ShapeDtypeStruct((K, D), jnp.float32),
    )(datat, centroid, idx8)


def kernel(datapoints, batch_cos_sim, centroid):
    del batch_cos_sim
    idx2d = _assign(datapoints, centroid)
    idx1d = jnp.reshape(idx2d, (B,))
    dp_cluster = _cluster(idx2d, idx1d)
    dpt = jnp.transpose(datapoints, (1, 0))
    zeros = jnp.zeros((NCOLS, K), jnp.float32)
    datat = _sc_scatter(dpt, idx1d, zeros)
    new_centroid = _final(datat, centroid, jnp.reshape(idx2d, (B // 512, 512)))
    return (new_centroid, dp_cluster)


# speedup vs baseline: 2.4871x; 2.4871x over previous
import functools

import jax
import jax.numpy as jnp
from jax import lax
from jax.experimental import pallas as pl
from jax.experimental.pallas import tpu as pltpu
from jax.experimental.pallas import tpu_sc as plsc

K = 1024
D = 768
B = 4096
TEMP = 1.0
LR = 0.001

BM = 256
BC = 1024

N_CORES = 2
N_SUB = 16
NW = N_CORES * N_SUB
NCOLS = D // NW
RCHUNK = 1024


def _assign_body(dp_ref, c_ref, idx_ref):
    c = c_ref[...]
    dp = dp_ref[...]
    dots = lax.dot_general(dp.astype(jnp.bfloat16), c.astype(jnp.bfloat16),
                           (((1,), (1,)), ((), ())),
                           preferred_element_type=jnp.float32)
    cn2 = lax.dot_general(jnp.ones((8, D), jnp.float32), c * c,
                          (((1,), (1,)), ((), ())),
                          preferred_element_type=jnp.float32,
                          precision=lax.Precision.HIGHEST)
    xn2 = jnp.sum(dp * dp, axis=1, keepdims=True)
    q = xn2 * cn2[0:1, :]
    r = lax.rsqrt(q)
    r = r * (1.5 - 0.5 * q * r * r)
    r = r * (1.5 - 0.5 * q * r * r)
    scale = jnp.where(q < 1e-16, 1e8, r) / TEMP
    score = dots * scale
    m = jnp.max(score, axis=1, keepdims=True)
    iot = lax.broadcasted_iota(jnp.int32, (BM, K), 1)
    cand = jnp.where(score == m, iot, K)
    idx_ref[...] = jnp.min(cand, axis=1, keepdims=True)


def _cluster_body(rows_ref, cols_ref, out_ref):
    bi = pl.program_id(0)
    bj = pl.program_id(1)
    r = rows_ref[...]
    cvec = cols_ref[...]
    eq = jnp.where(r == cvec[None, :], 1.0, 0.0)

    @pl.when(bi == bj)
    def _():
        ii = lax.broadcasted_iota(jnp.int32, (BC, BC), 0)
        jj = lax.broadcasted_iota(jnp.int32, (BC, BC), 1)
        out_ref[...] = jnp.where(ii == jj, 0.0, eq)

    @pl.when(bi != bj)
    def _():
        out_ref[...] = eq


def _sc_scatter_body(dpt_hbm, idx_hbm, zero_hbm, datat_hbm,
                     idx_v, rows_a, rows_b, acc, sem_a, sem_b):
    wid = lax.axis_index("c") * N_SUB + lax.axis_index("s")
    c0 = wid * NCOLS
    pltpu.sync_copy(zero_hbm.at[:], acc)
    pltpu.sync_copy(idx_hbm.at[:], idx_v)

    bufs = (rows_a, rows_b)
    sems = (sem_a, sem_b)
    nchunks = B // RCHUNK
    cps = {}
    cps[0] = pltpu.async_copy(
        dpt_hbm.at[pl.ds(c0, NCOLS), pl.ds(0, RCHUNK)], bufs[0], sems[0])
    for chunk in range(nchunks):
        base = chunk * RCHUNK
        if chunk + 1 < nchunks:
            cps[chunk + 1] = pltpu.async_copy(
                dpt_hbm.at[pl.ds(c0, NCOLS), pl.ds(base + RCHUNK, RCHUNK)],
                bufs[(chunk + 1) % 2], sems[(chunk + 1) % 2])
        cps[chunk].wait()
        cur = bufs[chunk % 2]

        def _group(g, _):
            kvec = idx_v[pl.ds(base + g * 16, 16)]
            for cc in range(NCOLS):
                vals = cur[cc, pl.ds(g * 16, 16)]
                plsc.addupdate_scatter(acc, [kvec + (cc * K)], vals)
            return 0

        lax.fori_loop(0, RCHUNK // 16, _group, 0)
    pltpu.sync_copy(acc, datat_hbm.at[pl.ds(c0 * K, NCOLS * K)])


def _final_body(datat_ref, c_ref, idx_ref, out_ref):
    iota_k = lax.broadcasted_iota(jnp.int32, (K, 512), 0)
    cnt = jnp.zeros((K, 1), jnp.float32)
    for j in range(B // 512):
        row = idx_ref[j, :]
        eq = jnp.where(iota_k == row[None, :], 1.0, 0.0)
        cnt = cnt + jnp.sum(eq, axis=1, keepdims=True)
    coef = (1.0 - LR) + LR * jnp.where(cnt == 0.0, 1.0, 0.0)
    data = jnp.transpose(datat_ref[...], (1, 0))
    out_ref[...] = LR * data + coef * c_ref[...]


def _assign(datapoints, centroid):
    return pl.pallas_call(
        _assign_body,
        grid=(B // BM,),
        in_specs=[pl.BlockSpec((BM, D), lambda i: (i, 0)),
                  pl.BlockSpec((K, D), lambda i: (0, 0))],
        out_specs=pl.BlockSpec((BM, 1), lambda i: (i, 0)),
        out_shape=jax.ShapeDtypeStruct((B, 1), jnp.int32),
    )(datapoints, centroid)


def _cluster(idx2d, idx1d):
    return pl.pallas_call(
        _cluster_body,
        grid=(B // BC, B // BC),
        in_specs=[pl.BlockSpec((BC, 1), lambda i, j: (i, 0)),
                  pl.BlockSpec((BC,), lambda i, j: (j,))],
        out_specs=pl.BlockSpec((BC, BC), lambda i, j: (i, j)),
        out_shape=jax.ShapeDtypeStruct((B, B), jnp.float32),
    )(idx2d, idx1d)


@functools.partial(
    pl.kernel,
    out_type=jax.ShapeDtypeStruct((D * K,), jnp.float32),
    mesh=plsc.VectorSubcoreMesh(core_axis_name="c", subcore_axis_name="s",
                                num_cores=N_CORES, num_subcores=N_SUB),
    compiler_params=pltpu.CompilerParams(use_tc_tiling_on_sc=False,
                                         needs_layout_passes=False),
    scratch_types=[
        pltpu.VMEM((B,), jnp.int32),
        pltpu.VMEM((NCOLS, RCHUNK), jnp.float32),
        pltpu.VMEM((NCOLS, RCHUNK), jnp.float32),
        pltpu.VMEM((NCOLS * K,), jnp.float32),
        pltpu.SemaphoreType.DMA,
        pltpu.SemaphoreType.DMA,
    ],
)
def _sc_scatter(dpt_hbm, idx_hbm, zero_hbm, datat_hbm, *rest):
    _sc_scatter_body(dpt_hbm, idx_hbm, zero_hbm, datat_hbm, *rest)


def _final(datat, centroid, idx8):
    return pl.pallas_call(
        _final_body,
        in_specs=[pl.BlockSpec((D, K), lambda: (0, 0)),
                  pl.BlockSpec((K, D), lambda: (0, 0)),
                  pl.BlockSpec((B // 512, 512), lambda: (0, 0))],
        out_specs=pl.BlockSpec((K, D), lambda: (0, 0)),
        out_shape=jax.ShapeDtypeStruct((K, D), jnp.float32),
    )(datat, centroid, idx8)


def kernel(datapoints, batch_cos_sim, centroid):
    del batch_cos_sim
    idx2d = _assign(datapoints, centroid)
    idx1d = jnp.reshape(idx2d, (B,))
    dp_cluster = _cluster(idx2d, idx1d)
    dpt = jnp.transpose(datapoints, (1, 0))
    zeros = jnp.zeros((NCOLS * K,), jnp.float32)
    datat = _sc_scatter(dpt, idx1d, zeros)
    new_centroid = _final(jnp.reshape(datat, (D, K)), centroid,
                          jnp.reshape(idx2d, (B // 512, 512)))
    return (new_centroid, dp_cluster)


# speedup vs baseline: 3.1464x; 1.2651x over previous
import functools

import jax
import jax.numpy as jnp
from jax import lax
from jax.experimental import pallas as pl
from jax.experimental.pallas import tpu as pltpu
from jax.experimental.pallas import tpu_sc as plsc

K = 1024
D = 768
B = 4096
TEMP = 1.0
LR = 0.001

BM = 512
BC = 1024

N_CORES = 2
N_SUB = 16
NW = N_CORES * N_SUB
NCOLS = D // NW
RSTAGE = 6


def _assign_body(dp_ref, cbf_ref, c_ref, idx_ref, cn2_ref):
    i = pl.program_id(0)

    @pl.when(i == 0)
    def _():
        c = c_ref[...]
        cn2_ref[...] = lax.dot_general(
            jnp.ones((8, D), jnp.float32), c * c,
            (((1,), (1,)), ((), ())),
            preferred_element_type=jnp.float32,
            precision=lax.Precision.HIGHEST)

    dp = dp_ref[...]
    dots = lax.dot_general(dp.astype(jnp.bfloat16), cbf_ref[...],
                           (((1,), (1,)), ((), ())),
                           preferred_element_type=jnp.float32)
    xn2 = jnp.sum(dp * dp, axis=1, keepdims=True)
    q = xn2 * cn2_ref[0:1, :]
    r = lax.rsqrt(q)
    r = r * (1.5 - 0.5 * q * r * r)
    r = r * (1.5 - 0.5 * q * r * r)
    scale = jnp.where(q < 1e-16, 1e8, r) / TEMP
    score = dots * scale
    m = jnp.max(score, axis=1, keepdims=True)
    iot = lax.broadcasted_iota(jnp.int32, (BM, K), 1)
    cand = jnp.where(score == m, iot, K)
    idx_ref[...] = jnp.min(cand, axis=1, keepdims=True)


def _cluster_body(rows_ref, cols_ref, out_ref):
    bi = pl.program_id(0)
    bj = pl.program_id(1)
    r = rows_ref[...]
    cvec = cols_ref[...]
    eq = jnp.where(r == cvec[None, :], 1.0, 0.0)

    @pl.when(bi == bj)
    def _():
        ii = lax.broadcasted_iota(jnp.int32, (BC, BC), 0)
        jj = lax.broadcasted_iota(jnp.int32, (BC, BC), 1)
        out_ref[...] = jnp.where(ii == jj, 0.0, eq)

    @pl.when(bi != bj)
    def _():
        out_ref[...] = eq


def _sc_scatter_body(dpt_hbm, idx_hbm, zero_hbm, datat_hbm,
                     idx_v, rows_a, rows_b, acc, sem_a, sem_b):
    wid = lax.axis_index("c") * N_SUB + lax.axis_index("s")
    c0 = wid * NCOLS
    pltpu.sync_copy(zero_hbm.at[:], acc)
    pltpu.sync_copy(idx_hbm.at[:], idx_v)

    bufs = (rows_a, rows_b)
    sems = (sem_a, sem_b)
    nstages = NCOLS // RSTAGE
    cps = {}
    cps[0] = pltpu.async_copy(
        dpt_hbm.at[pl.ds(c0 * B, RSTAGE * B)], bufs[0], sems[0])
    for s in range(nstages):
        if s + 1 < nstages:
            cps[s + 1] = pltpu.async_copy(
                dpt_hbm.at[pl.ds((c0 + (s + 1) * RSTAGE) * B, RSTAGE * B)],
                bufs[(s + 1) % 2], sems[(s + 1) % 2])
        cps[s].wait()
        cur = bufs[s % 2]

        def _group(g, _):
            kvec = idx_v[pl.ds(g * 16, 16)]
            for cc in range(RSTAGE):
                vals = cur[pl.ds(cc * B + g * 16, 16)]
                plsc.addupdate_scatter(acc, [kvec + ((s * RSTAGE + cc) * K)],
                                       vals)
            return 0

        lax.fori_loop(0, B // 16, _group, 0)
    pltpu.sync_copy(acc, datat_hbm.at[pl.ds(c0 * K, NCOLS * K)])


def _final_body(datat_ref, c_ref, idx_ref, out_ref):
    iota_k = lax.broadcasted_iota(jnp.int32, (K, 512), 0)
    cnt = jnp.zeros((K, 1), jnp.float32)
    for j in range(B // 512):
        row = idx_ref[j, :]
        eq = jnp.where(iota_k == row[None, :], 1.0, 0.0)
        cnt = cnt + jnp.sum(eq, axis=1, keepdims=True)
    coef = (1.0 - LR) + LR * jnp.where(cnt == 0.0, 1.0, 0.0)
    data = jnp.transpose(datat_ref[...], (1, 0))
    out_ref[...] = LR * data + coef * c_ref[...]


def _assign(datapoints, centroid_bf, centroid):
    return pl.pallas_call(
        _assign_body,
        grid=(B // BM,),
        in_specs=[pl.BlockSpec((BM, D), lambda i: (i, 0)),
                  pl.BlockSpec((K, D), lambda i: (0, 0)),
                  pl.BlockSpec((K, D), lambda i: (0, 0))],
        out_specs=pl.BlockSpec((BM, 1), lambda i: (i, 0)),
        out_shape=jax.ShapeDtypeStruct((B, 1), jnp.int32),
        scratch_shapes=[pltpu.VMEM((8, K), jnp.float32)],
    )(datapoints, centroid_bf, centroid)


def _cluster(idx2d, idx1d):
    return pl.pallas_call(
        _cluster_body,
        grid=(B // BC, B // BC),
        in_specs=[pl.BlockSpec((BC, 1), lambda i, j: (i, 0)),
                  pl.BlockSpec((BC,), lambda i, j: (j,))],
        out_specs=pl.BlockSpec((BC, BC), lambda i, j: (i, j)),
        out_shape=jax.ShapeDtypeStruct((B, B), jnp.float32),
    )(idx2d, idx1d)


@functools.partial(
    pl.kernel,
    out_type=jax.ShapeDtypeStruct((D * K,), jnp.float32),
    mesh=plsc.VectorSubcoreMesh(core_axis_name="c", subcore_axis_name="s",
                                num_cores=N_CORES, num_subcores=N_SUB),
    compiler_params=pltpu.CompilerParams(use_tc_tiling_on_sc=False,
                                         needs_layout_passes=False),
    scratch_types=[
        pltpu.VMEM((B,), jnp.int32),
        pltpu.VMEM((RSTAGE * B,), jnp.float32),
        pltpu.VMEM((RSTAGE * B,), jnp.float32),
        pltpu.VMEM((NCOLS * K,), jnp.float32),
        pltpu.SemaphoreType.DMA,
        pltpu.SemaphoreType.DMA,
    ],
)
def _sc_scatter(dpt_hbm, idx_hbm, zero_hbm, datat_hbm, *rest):
    _sc_scatter_body(dpt_hbm, idx_hbm, zero_hbm, datat_hbm, *rest)


def _final(datat, centroid, idx8):
    return pl.pallas_call(
        _final_body,
        in_specs=[pl.BlockSpec((D, K), lambda: (0, 0)),
                  pl.BlockSpec((K, D), lambda: (0, 0)),
                  pl.BlockSpec((B // 512, 512), lambda: (0, 0))],
        out_specs=pl.BlockSpec((K, D), lambda: (0, 0)),
        out_shape=jax.ShapeDtypeStruct((K, D), jnp.float32),
    )(datat, centroid, idx8)


def kernel(datapoints, batch_cos_sim, centroid):
    del batch_cos_sim
    centroid_bf = centroid.astype(jnp.bfloat16)
    idx2d = _assign(datapoints, centroid_bf, centroid)
    idx1d = jnp.reshape(idx2d, (B,))
    dpt = jnp.reshape(jnp.transpose(datapoints, (1, 0)), (D * B,))
    zeros = jnp.zeros((NCOLS * K,), jnp.float32)
    datat = _sc_scatter(dpt, idx1d, zeros)
    dp_cluster = _cluster(idx2d, idx1d)
    new_centroid = _final(jnp.reshape(datat, (D, K)), centroid,
                          jnp.reshape(idx2d, (B // 512, 512)))
    return (new_centroid, dp_cluster)
